# baseline (speedup 1.0000x reference)
import functools

import jax
import jax.numpy as jnp
from jax import lax
from jax.experimental import pallas as pl
from jax.experimental.pallas import tpu as pltpu
from jax.experimental.pallas import tpu_sc as plsc

NC, NS = 2, 16
K = 128
PAD = {"OER": 55296, "Concept": 10240, "Class": 4096}
NR = 6
RSZ = 9216
F32 = jnp.float32

_LONG = {"ep": "before_ep", "cov": "covers", "bel": "belongs",
         "rcov": "rev_covers", "rbel": "rev_belongs"}


def _rpad(x, n):
    return jnp.pad(x, ((0, n - x.shape[0]),) + ((0, 0),) * (x.ndim - 1))


def _mm_bias(x, w, b, bn=1024):
    n, k = x.shape

    def body(x_ref, w_ref, b_ref, o_ref):
        o_ref[...] = jnp.dot(x_ref[...], w_ref[...],
                             preferred_element_type=F32) + b_ref[0]

    return pl.pallas_call(
        body,
        grid=(n // bn,),
        in_specs=[pl.BlockSpec((bn, k), lambda i: (i, 0)),
                  pl.BlockSpec((k, 128), lambda i: (0, 0)),
                  pl.BlockSpec((1, 128), lambda i: (0, 0))],
        out_specs=pl.BlockSpec((bn, 128), lambda i: (i, 0)),
        out_shape=jax.ShapeDtypeStruct((n, 128), F32),
    )(x, w, b.reshape(1, 128))


def _edge_w_kernel(n_dst_pad, E, E_pad):
    stripe_e = E_pad // (NC * NS)
    stripe_n = n_dst_pad // NS
    mesh = plsc.VectorSubcoreMesh(core_axis_name="c", subcore_axis_name="s")

    @functools.partial(
        pl.kernel,
        out_type=(jax.ShapeDtypeStruct((E_pad,), F32),
                  jax.ShapeDtypeStruct((E_pad,), F32),
                  jax.ShapeDtypeStruct((NC * 2 * n_dst_pad,), F32)),
        mesh=mesh,
        scratch_types=[
            pltpu.VMEM((K,), jnp.int32), pltpu.VMEM((K,), jnp.int32),
            pltpu.VMEM((K,), F32), pltpu.VMEM((K,), F32),
            pltpu.VMEM((K,), F32), pltpu.VMEM((K,), F32),
            pltpu.VMEM((K,), F32), pltpu.VMEM((K,), F32),
            pltpu.VMEM_SHARED((n_dst_pad,), F32),
            pltpu.VMEM_SHARED((n_dst_pad,), F32),
            pltpu.VMEM((stripe_n,), F32),
            pltpu.SemaphoreType.DMA,
        ],
    )
    def kern(as0, as1, ad0, ad1, src, dst, zeros1d, w0_out, w1_out, s_out,
             srcv, dstv, a0v, a1v, b0v, b1v, w0v, w1v,
             s0_sh, s1_sh, bounce, sem):
        c = lax.axis_index("c")
        s = lax.axis_index("s")
        wid = c * NS + s
        pltpu.sync_copy(zeros1d.at[pl.ds(0, stripe_n)], bounce)
        pltpu.sync_copy(bounce, s0_sh.at[pl.ds(s * stripe_n, stripe_n)])
        pltpu.sync_copy(bounce, s1_sh.at[pl.ds(s * stripe_n, stripe_n)])
        plsc.subcore_barrier()

        @pl.loop(0, stripe_e // K)
        def chunk(t):
            base = wid * stripe_e + t * K
            pltpu.sync_copy(src.at[pl.ds(base, K)], srcv)
            pltpu.sync_copy(dst.at[pl.ds(base, K)], dstv)
            pltpu.async_copy(as0.at[srcv], a0v, sem).wait()
            pltpu.async_copy(as1.at[srcv], a1v, sem).wait()
            pltpu.async_copy(ad0.at[dstv], b0v, sem).wait()
            pltpu.async_copy(ad1.at[dstv], b1v, sem).wait()
            for j in range(K // 16):
                sl = pl.ds(j * 16, 16)
                pos = base + j * 16 + lax.iota(jnp.int32, 16)
                valid = pos < E
                x0 = a0v[sl] + b0v[sl]
                x1 = a1v[sl] + b1v[sl]
                w0 = jnp.exp(jnp.maximum(x0, 0.2 * x0))
                w1 = jnp.exp(jnp.maximum(x1, 0.2 * x1))
                w0v[sl] = jnp.where(valid, w0, 0.0)
                w1v[sl] = jnp.where(valid, w1, 0.0)
            pltpu.sync_copy(w0v, w0_out.at[pl.ds(base, K)])
            pltpu.sync_copy(w1v, w1_out.at[pl.ds(base, K)])
            pltpu.sync_copy(w0v, s0_sh.at[dstv], add=True)
            pltpu.sync_copy(w1v, s1_sh.at[dstv], add=True)

        plsc.subcore_barrier()
        r0 = s * stripe_n
        pltpu.sync_copy(s0_sh.at[pl.ds(r0, stripe_n)], bounce)
        pltpu.sync_copy(bounce,
                        s_out.at[pl.ds((c * 2 + 0) * n_dst_pad + r0, stripe_n)])
        pltpu.sync_copy(s1_sh.at[pl.ds(r0, stripe_n)], bounce)
        pltpu.sync_copy(bounce,
                        s_out.at[pl.ds((c * 2 + 1) * n_dst_pad + r0, stripe_n)])

    return kern


KB = 512


def _edge_acc_kernel(n_dst_pad, E_pad, ranged):
    mesh = plsc.VectorSubcoreMesh(core_axis_name="c", subcore_axis_name="s")
    if ranged:
        acc_rows = RSZ
        out_ty = jax.ShapeDtypeStruct((n_dst_pad, 128), F32)
        stripe_e = E_pad // NS
    else:
        acc_rows = n_dst_pad
        out_ty = jax.ShapeDtypeStruct((NC, n_dst_pad, 128), F32)
        stripe_e = E_pad // (NC * NS)
    zstripe = acc_rows // NS

    scratch = [
        pltpu.VMEM((KB,), jnp.int32), pltpu.VMEM((KB,), jnp.int32),
        pltpu.VMEM((KB,), F32), pltpu.VMEM((KB,), F32),
        pltpu.VMEM((K,), jnp.int32), pltpu.VMEM((K,), jnp.int32),
        pltpu.VMEM((K,), F32), pltpu.VMEM((K,), F32),
        pltpu.VMEM((K,), F32), pltpu.VMEM((K,), F32),
        pltpu.VMEM((K, 128), F32), pltpu.VMEM((K, 128), F32),
        pltpu.VMEM((K, 128), F32),
        pltpu.VMEM_SHARED((acc_rows, 128), F32),
        pltpu.VMEM((32, 128), F32),
        pltpu.SemaphoreType.DMA, pltpu.SemaphoreType.DMA,
        pltpu.SemaphoreType.DMA,
    ]

    @functools.partial(pl.kernel, out_type=out_ty, mesh=mesh,
                       scratch_types=scratch)
    def kern(xs, src, dst, w0, w1, out,
             srcv, dstv, w0v, w1v, ldst0, ldst1, mw00, mw01, mw10, mw11,
             rows0, rows1, msg, acc_sh, zbuf, seml, semg0, semg1):
        c = lax.axis_index("c")
        s = lax.axis_index("s")
        rowbufs = (rows0, rows1)
        ldsts = (ldst0, ldst1)
        mw0s = (mw00, mw01)
        mw1s = (mw10, mw11)
        gsems = (semg0, semg1)

        @pl.loop(0, 32)
        def zfill(r):
            for j in range(8):
                zbuf[r, pl.ds(j * 16, 16)] = jnp.zeros((16,), F32)

        def run_pass(base_row, ebase, out_view):
            @pl.loop(0, zstripe // 32)
            def zrow(zz):
                pltpu.sync_copy(
                    zbuf, acc_sh.at[pl.ds(s * zstripe + zz * 32, 32), :])

            plsc.subcore_barrier()

            @pl.loop(0, stripe_e // KB)
            def blk(tb):
                base = ebase + s * stripe_e + tb * KB
                d1 = pltpu.async_copy(src.at[pl.ds(base, KB)], srcv, seml)
                d2 = pltpu.async_copy(dst.at[pl.ds(base, KB)], dstv, seml)
                d3 = pltpu.async_copy(w0.at[pl.ds(base, KB)], w0v, seml)
                d4 = pltpu.async_copy(w1.at[pl.ds(base, KB)], w1v, seml)
                d1.wait(); d2.wait(); d3.wait(); d4.wait()
                nsub = KB // K
                if not ranged:
                    @pl.loop(0, nsub)
                    def sub(j):
                        gd = pltpu.async_copy(
                            xs.at[srcv.at[pl.ds(j * K, K)]], rows0, semg0)
                        for g in range(K // 16):
                            sl = pl.ds(g * 16, 16)
                            ldst0[sl] = dstv[pl.ds(j * K + g * 16, 16)]
                        gd.wait()

                        @pl.loop(0, K // 16)
                        def edge_grp(g):
                            w0vec = w0v[pl.ds(j * K + g * 16, 16)]
                            w1vec = w1v[pl.ds(j * K + g * 16, 16)]
                            for e16 in range(16):
                                e = g * 16 + e16
                                ws0 = w0vec[e16]
                                ws1 = w1vec[e16]
                                for jj in range(4):
                                    slc = pl.ds(jj * 16, 16)
                                    msg[e, slc] = rows0[e, slc] * ws0
                                for jj in range(4, 8):
                                    slc = pl.ds(jj * 16, 16)
                                    msg[e, slc] = rows0[e, slc] * ws1

                        pltpu.sync_copy(msg, acc_sh.at[ldst0], add=True)

                    return
                gds = [None] * nsub
                gds[0] = pltpu.async_copy(
                    xs.at[srcv.at[pl.ds(0, K)]], rowbufs[0], gsems[0])
                for j in range(nsub):
                    nb = j % 2
                    if j + 1 < nsub:
                        gds[j + 1] = pltpu.async_copy(
                            xs.at[srcv.at[pl.ds((j + 1) * K, K)]],
                            rowbufs[1 - nb], gsems[1 - nb])
                    for g in range(K // 16):
                        sl = pl.ds(g * 16, 16)
                        slk = pl.ds(j * K + g * 16, 16)
                        dv = dstv[slk]
                        lv = dv - base_row
                        ok = (lv >= 0) & (lv < RSZ)
                        ldsts[nb][sl] = jnp.where(ok, lv, 0)
                        mw0s[nb][sl] = jnp.where(ok, w0v[slk], 0.0)
                        mw1s[nb][sl] = jnp.where(ok, w1v[slk], 0.0)
                    gds[j].wait()
                    rows = rowbufs[nb]

                    @pl.loop(0, K // 16)
                    def edge_grp(g):
                        w0vec = mw0s[nb][pl.ds(g * 16, 16)]
                        w1vec = mw1s[nb][pl.ds(g * 16, 16)]
                        for e16 in range(16):
                            e = g * 16 + e16
                            ws0 = w0vec[e16]
                            ws1 = w1vec[e16]
                            for jj in range(4):
                                slc = pl.ds(jj * 16, 16)
                                msg[e, slc] = rows[e, slc] * ws0
                            for jj in range(4, 8):
                                slc = pl.ds(jj * 16, 16)
                                msg[e, slc] = rows[e, slc] * ws1

                    pltpu.sync_copy(msg, acc_sh.at[ldsts[nb]], add=True)

            plsc.subcore_barrier()
            per_tile = (RSZ if ranged else n_dst_pad) // NS

            @pl.loop(0, per_tile // 32)
            def dmp(tt):
                r0 = s * per_tile + tt * 32
                pltpu.sync_copy(acc_sh.at[pl.ds(r0, 32), :],
                                msg.at[pl.ds(0, 32), :])
                pltpu.sync_copy(msg.at[pl.ds(0, 32), :],
                                out_view.at[pl.ds(base_row + r0, 32), :])

            plsc.subcore_barrier()

        if ranged:
            @pl.loop(0, NR // NC)
            def rng(rr):
                run_pass((c * (NR // NC) + rr) * RSZ, 0, out)
        else:
            run_pass(0, c * (NS * stripe_e), out.at[c])

    return kern


def _edge_cls_kernel(E_pad):
    stripe_e = E_pad // (NC * NS)
    mesh = plsc.VectorSubcoreMesh(core_axis_name="c", subcore_axis_name="s")

    @functools.partial(
        pl.kernel,
        out_type=jax.ShapeDtypeStruct((E_pad,), F32),
        mesh=mesh,
        scratch_types=[
            pltpu.VMEM((K,), jnp.int32), pltpu.VMEM((K,), jnp.int32),
            pltpu.VMEM((K,), F32), pltpu.VMEM((K,), F32),
            pltpu.VMEM((K,), F32),
            pltpu.SemaphoreType.DMA,
        ],
    )
    def kern(u1, u2, src, dst, out, srcv, dstv, g1, g2, ov, sem):
        c = lax.axis_index("c")
        s = lax.axis_index("s")
        wid = c * NS + s

        @pl.loop(0, stripe_e // K)
        def chunk(t):
            base = wid * stripe_e + t * K
            pltpu.sync_copy(src.at[pl.ds(base, K)], srcv)
            pltpu.sync_copy(dst.at[pl.ds(base, K)], dstv)
            pltpu.async_copy(u1.at[srcv], g1, sem).wait()
            pltpu.async_copy(u2.at[dstv], g2, sem).wait()
            for j in range(K // 16):
                sl = pl.ds(j * 16, 16)
                ov[sl] = g1[sl] + g2[sl]
            pltpu.sync_copy(ov, out.at[pl.ds(base, K)])

    return kern


def _finalize(gats, n_pad, bn=1024):
    navg = 1.0 / len(gats)
    specs, args, has_self, split_acc = [], [], [], []
    for g in gats:
        if g["acc"].ndim == 3:
            specs.append(pl.BlockSpec((NC, bn, 128), lambda i: (0, i, 0)))
            split_acc.append(True)
        else:
            specs.append(pl.BlockSpec((bn, 128), lambda i: (i, 0)))
            split_acc.append(False)
        specs += [pl.BlockSpec((NC, 2, bn), lambda i: (0, 0, i)),
                  pl.BlockSpec((1, 128), lambda i: (0, 0))]
        args += [g["acc"], g["sp"], g["bias"].reshape(1, 128)]
        has_self.append("xs" in g)
        if "xs" in g:
            specs.append(pl.BlockSpec((bn, 128), lambda i: (i, 0)))
            args.append(g["xs"])
            for t in ("as0", "as1", "ad0", "ad1"):
                specs.append(pl.BlockSpec((1, bn), lambda i: (0, i)))
                args.append(g[t])

    def body(*refs):
        o_ref = refs[-1]
        refs = list(refs[:-1])
        total = None
        for self_l, sp_acc in zip(has_self, split_acc):
            acc_ref, sp_ref, b_ref = refs[:3]
            del refs[:3]
            num = acc_ref[0] + acc_ref[1] if sp_acc else acc_ref[...]
            s0 = sp_ref[0, 0] + sp_ref[1, 0]
            s1 = sp_ref[0, 1] + sp_ref[1, 1]
            if self_l:
                xs_ref, as0, as1, ad0, ad1 = refs[:5]
                del refs[:5]
                x0 = as0[0] + ad0[0]
                x1 = as1[0] + ad1[0]
                w0 = jnp.exp(jnp.maximum(x0, 0.2 * x0))
                w1 = jnp.exp(jnp.maximum(x1, 0.2 * x1))
                wcat = jnp.concatenate(
                    [jnp.broadcast_to(w0[:, None], (bn, 64)),
                     jnp.broadcast_to(w1[:, None], (bn, 64))], axis=-1)
                num = num + wcat * xs_ref[...]
                s0 = s0 + w0
                s1 = s1 + w1
            den = jnp.concatenate(
                [jnp.broadcast_to((s0 + 1e-16)[:, None], (bn, 64)),
                 jnp.broadcast_to((s1 + 1e-16)[:, None], (bn, 64))], axis=-1)
            part = num / den + b_ref[0]
            total = part if total is None else total + part
        o_ref[...] = total * navg

    return pl.pallas_call(
        body,
        grid=(n_pad // bn,),
        in_specs=specs,
        out_specs=pl.BlockSpec((bn, 128), lambda i: (i, 0)),
        out_shape=jax.ShapeDtypeStruct((n_pad, 128), F32),
    )(*args)


def _acol(p, which, h):
    W = p["Wsrc" if which == "s" else "Wdst"]
    att = p["att_src" if which == "s" else "att_dst"][h]
    col = W[:, h * 64:(h + 1) * 64] @ att
    return jnp.pad(col, (0, 128 - col.shape[0]))


def _wpad(W):
    return jnp.pad(W, ((0, 128 - W.shape[0]), (0, 0)))


def kernel(x_oer, x_concept, x_class, params, ei_sr, ei_ep, ei_cov, ei_bel,
           ei_rcov, ei_rbel):
    NPo, NPc, NPk = PAD["OER"], PAD["Concept"], PAD["Class"]
    zeros1d = jnp.zeros((NPo // NS,), F32)
    zero_b = jnp.zeros((128,), F32)

    def pad_edges(ei, mult=16384):
        E = ei.shape[1]
        E_pad = -(-E // mult) * mult
        ei = jnp.pad(ei, ((0, 0), (0, E_pad - E)))
        return ei[0], ei[1], E, E_pad

    edges = {
        "ep": pad_edges(ei_ep) + ("OER", "OER"),
        "cov": pad_edges(ei_cov) + ("OER", "Concept"),
        "bel": pad_edges(ei_bel) + ("Concept", "Class"),
        "rcov": pad_edges(ei_rcov) + ("Concept", "OER"),
        "rbel": pad_edges(ei_rbel) + ("Class", "Concept"),
    }

    x_pad = {"OER": _rpad(x_oer, NPo), "Concept": _rpad(x_concept, NPc),
             "Class": _rpad(x_class, NPk)}
    h = {}
    for nt in ("OER", "Concept", "Class"):
        W0 = jnp.pad(params["lin"][nt]["W"], ((0, 0), (0, 64)))
        b0 = jnp.pad(params["lin"][nt]["b"], (0, 64))
        h[nt] = _mm_bias(x_pad[nt], W0, b0)

    acol_cols = {
        "OER": [("ep", "s"), ("ep", "d"), ("cov", "s"), ("rcov", "d")],
        "Concept": [("cov", "d"), ("bel", "s"), ("rcov", "s"), ("rbel", "d")],
        "Class": [("bel", "d"), ("rbel", "s")],
    }

    for lp in params["layers"]:
        xs = {name: _mm_bias(h[st], _wpad(lp[_LONG[name]]["Wsrc"]), zero_b)
              for name, (_, _, _, _, st, _) in edges.items()}
        atab = {}
        for nt, colspec in acol_cols.items():
            cols = []
            for gname, which in colspec:
                cols += [_acol(lp[_LONG[gname]], which, 0),
                         _acol(lp[_LONG[gname]], which, 1)]
            wa = jnp.stack(cols + [jnp.zeros((128,), F32)] *
                           (128 - len(cols)), axis=1)
            am = _mm_bias(h[nt], wa, zero_b)
            for i2, (gname, which) in enumerate(colspec):
                atab[(gname, which, 0)] = am[:, 2 * i2]
                atab[(gname, which, 1)] = am[:, 2 * i2 + 1]

        res = {}
        for name, (src, dst, E, E_pad, st, dt) in edges.items():
            n_dst_pad = PAD[dt]
            wk = _edge_w_kernel(n_dst_pad, E, E_pad)
            w0_e, w1_e, s_part = wk(atab[(name, "s", 0)], atab[(name, "s", 1)],
                                    atab[(name, "d", 0)], atab[(name, "d", 1)],
                                    src, dst, zeros1d)
            ak = _edge_acc_kernel(n_dst_pad, E_pad, ranged=(dt == "OER"))
            acc = ak(xs[name], src, dst, w0_e, w1_e)
            res[name] = {"acc": acc, "sp": s_part.reshape(NC, 2, n_dst_pad),
                         "bias": lp[_LONG[name]]["bias"]}

        res["ep"]["xs"] = xs["ep"]
        res["ep"]["as0"] = atab[("ep", "s", 0)].reshape(1, -1)
        res["ep"]["as1"] = atab[("ep", "s", 1)].reshape(1, -1)
        res["ep"]["ad0"] = atab[("ep", "d", 0)].reshape(1, -1)
        res["ep"]["ad1"] = atab[("ep", "d", 1)].reshape(1, -1)

        h = {
            "OER": _finalize([res["ep"], res["rcov"]], NPo),
            "Concept": _finalize([res["cov"], res["rbel"]], NPc),
            "Class": _finalize([res["bel"]], NPk),
        }

    Wc = params["cls"]["W"]
    bc = params["cls"]["b"]
    wmat = jnp.zeros((256, 128), F32)
    wmat = wmat.at[:, 0].set(Wc[:256, 0]).at[:, 1].set(Wc[256:, 0])
    bvec = jnp.zeros((128,), F32).at[0].set(bc[0])
    xcat = jnp.concatenate([x_pad["OER"], h["OER"]], axis=1)
    u = _mm_bias(xcat, wmat, bvec)
    u1 = u[:, 0]
    u2 = u[:, 1]
    src, dst, E, E_pad = pad_edges(ei_sr)
    ck = _edge_cls_kernel(E_pad)
    pred = ck(u1, u2, src, dst)
    return pred[:E]
